# baseline (speedup 1.0000x reference)
import functools

import jax
import jax.numpy as jnp
from jax import lax
from jax.experimental import pallas as pl
from jax.experimental.pallas import tpu as pltpu
from jax.experimental.pallas import tpu_sc as plsc

LANES = 16
ROWS_PER_GROUP = 16
GROUPS_PER_CHUNK = 10
CHUNK_ROWS = GROUPS_PER_CHUNK * ROWS_PER_GROUP
NUM_WORKERS = 32


def _compute_chunk(xb, o0, o1, o2):
    e = xb.shape[-1]

    @pl.loop(0, GROUPS_PER_CHUNK)
    def _(g):
        r0 = g * ROWS_PER_GROUP
        for j in range(e // LANES):
            sl = pl.ds(j * LANES, LANES)
            means0 = []
            rows = []
            for s in range(4):
                r = [xb[r0 + 4 * s + i, sl] for i in range(4)]
                rows.append(r)
                means0.append(((r[0] + r[1]) + (r[2] + r[3])) * 0.25)
            for s in range(4):
                for i in range(4):
                    o0[r0 + 4 * s + i, sl] = rows[s][i] - means0[s]
            m1 = ((means0[0] + means0[1]) + (means0[2] + means0[3])) * 0.25
            for s in range(4):
                o1[g * 4 + s, sl] = means0[s] - m1
            o2[g, sl] = m1


def _decomp_sc(xc):
    n_chunks, cr, e = xc.shape
    mesh = plsc.VectorSubcoreMesh(
        core_axis_name="core",
        subcore_axis_name="subcore",
        num_cores=2,
        num_subcores=16,
    )

    @functools.partial(
        pl.kernel,
        out_type=(
            jax.ShapeDtypeStruct((n_chunks, cr, e), jnp.float32),
            jax.ShapeDtypeStruct((n_chunks, cr // 4, e), jnp.float32),
            jax.ShapeDtypeStruct((n_chunks, 16, e), jnp.float32),
        ),
        mesh=mesh,
        scratch_types=(
            pltpu.VMEM((cr, e), jnp.float32),
            pltpu.VMEM((cr, e), jnp.float32),
            pltpu.VMEM((cr // 4, e), jnp.float32),
            pltpu.VMEM((16, e), jnp.float32),
        ),
    )
    def run(x_hbm, o0_hbm, o1_hbm, o2_hbm, xb, o0b, o1b, o2b):
        wid = lax.axis_index("subcore") * 2 + lax.axis_index("core")

        @pl.loop(wid, n_chunks, step=NUM_WORKERS)
        def _(t):
            pltpu.sync_copy(x_hbm.at[t], xb)
            _compute_chunk(xb, o0b, o1b, o2b)
            pltpu.sync_copy(o0b, o0_hbm.at[t])
            pltpu.sync_copy(o1b, o1_hbm.at[t])
            pltpu.sync_copy(o2b, o2_hbm.at[t])

    return run(xc)


def _decomp_sc_pipelined(xc):
    n_chunks, cr, e = xc.shape
    mesh = plsc.VectorSubcoreMesh(
        core_axis_name="core",
        subcore_axis_name="subcore",
        num_cores=2,
        num_subcores=16,
    )

    def body(xb, o0, o1, o2):
        _compute_chunk(xb.at[0], o0.at[0], o1.at[0], o2.at[0])

    @functools.partial(
        pl.kernel,
        out_type=(
            jax.ShapeDtypeStruct((n_chunks, cr, e), jnp.float32),
            jax.ShapeDtypeStruct((n_chunks, cr // 4, e), jnp.float32),
            jax.ShapeDtypeStruct((n_chunks, 16, e), jnp.float32),
        ),
        mesh=mesh,
        scratch_types=(),
    )
    def run(x_hbm, o0_hbm, o1_hbm, o2_hbm):
        pltpu.emit_pipeline(
            body,
            grid=(n_chunks,),
            in_specs=[pl.BlockSpec((1, cr, e), lambda i: (i, 0, 0))],
            out_specs=[
                pl.BlockSpec((1, cr, e), lambda i: (i, 0, 0)),
                pl.BlockSpec((1, cr // 4, e), lambda i: (i, 0, 0)),
                pl.BlockSpec((1, 16, e), lambda i: (i, 0, 0)),
            ],
            core_axis_name=("core", "subcore"),
            dimension_semantics=(pltpu.PARALLEL,),
        )(x_hbm, o0_hbm, o1_hbm, o2_hbm)

    return run(xc)


def _compute_chunk_l1(xb, o1, o2):
    e = xb.shape[-1]

    @pl.loop(0, GROUPS_PER_CHUNK)
    def _(g):
        r0 = g * ROWS_PER_GROUP
        for j in range(e // LANES):
            sl = pl.ds(j * LANES, LANES)
            means0 = []
            for s in range(4):
                r = [xb[r0 + 4 * s + i, sl] for i in range(4)]
                means0.append(((r[0] + r[1]) + (r[2] + r[3])) * 0.25)
            m1 = ((means0[0] + means0[1]) + (means0[2] + means0[3])) * 0.25
            for s in range(4):
                o1[g * 4 + s, sl] = means0[s] - m1
            o2[g, sl] = m1


def _level1_sc(xc):
    n_chunks, cr, e = xc.shape
    mesh = plsc.VectorSubcoreMesh(
        core_axis_name="core",
        subcore_axis_name="subcore",
        num_cores=2,
        num_subcores=16,
    )

    def body(xb, o1, o2):
        _compute_chunk_l1(xb.at[0], o1.at[0], o2.at[0])

    @functools.partial(
        pl.kernel,
        out_type=(
            jax.ShapeDtypeStruct((n_chunks, cr // 4, e), jnp.float32),
            jax.ShapeDtypeStruct((n_chunks, 16, e), jnp.float32),
        ),
        mesh=mesh,
        scratch_types=(),
    )
    def run(x_hbm, o1_hbm, o2_hbm):
        pltpu.emit_pipeline(
            body,
            grid=(n_chunks,),
            in_specs=[pl.BlockSpec((1, cr, e), lambda i: (i, 0, 0))],
            out_specs=[
                pl.BlockSpec((1, cr // 4, e), lambda i: (i, 0, 0)),
                pl.BlockSpec((1, 16, e), lambda i: (i, 0, 0)),
            ],
            core_axis_name=("core", "subcore"),
            dimension_semantics=(pltpu.PARALLEL,),
        )(x_hbm, o1_hbm, o2_hbm)

    return run(xc)


TC_BLOCK_GROUPS = 625


def _residual0_tc(xg):
    n_groups, four, e = xg.shape
    bg = TC_BLOCK_GROUPS

    def body(x_ref, o_ref):
        xs = x_ref[...]
        m0 = jnp.mean(xs, axis=1, keepdims=True)
        o_ref[...] = xs - m0

    return pl.pallas_call(
        body,
        grid=(n_groups // bg,),
        in_specs=[pl.BlockSpec((bg, four, e), lambda i: (i, 0, 0))],
        out_specs=pl.BlockSpec((bg, four, e), lambda i: (i, 0, 0)),
        out_shape=jax.ShapeDtypeStruct((n_groups, four, e), jnp.float32),
    )(xg)


@jax.jit
def kernel(x, indices_level0, indices_level1, sample_dict=0):
    b, n, e = x.shape
    n_chunks = (b * n) // CHUNK_ROWS
    xc = x.reshape(n_chunks, CHUNK_ROWS, e)
    o1, o2p = _level1_sc(xc)
    o0 = _residual0_tc(x.reshape((b * n) // 4, 4, e))
    o2 = o2p[:, :GROUPS_PER_CHUNK, :]
    return (
        o0.reshape(b, n, e),
        o1.reshape(b, n // 4, e),
        o2.reshape(b, n // 16, e),
    )


# speedup vs baseline: 1.2813x; 1.2813x over previous
import functools

import jax
import jax.numpy as jnp
from jax import lax
from jax.experimental import pallas as pl
from jax.experimental.pallas import tpu as pltpu
from jax.experimental.pallas import tpu_sc as plsc

LANES = 16
ROWS_PER_GROUP = 16
GROUPS_PER_CHUNK = 10
CHUNK_ROWS = GROUPS_PER_CHUNK * ROWS_PER_GROUP
NUM_WORKERS = 32


def _compute_chunk(xb, o0, o1, o2):
    e = xb.shape[-1]

    @pl.loop(0, GROUPS_PER_CHUNK)
    def _(g):
        r0 = g * ROWS_PER_GROUP
        for j in range(e // LANES):
            sl = pl.ds(j * LANES, LANES)
            means0 = []
            rows = []
            for s in range(4):
                r = [xb[r0 + 4 * s + i, sl] for i in range(4)]
                rows.append(r)
                means0.append(((r[0] + r[1]) + (r[2] + r[3])) * 0.25)
            for s in range(4):
                for i in range(4):
                    o0[r0 + 4 * s + i, sl] = rows[s][i] - means0[s]
            m1 = ((means0[0] + means0[1]) + (means0[2] + means0[3])) * 0.25
            for s in range(4):
                o1[g * 4 + s, sl] = means0[s] - m1
            o2[g, sl] = m1


def _decomp_sc(xc):
    n_chunks, cr, e = xc.shape
    mesh = plsc.VectorSubcoreMesh(
        core_axis_name="core",
        subcore_axis_name="subcore",
        num_cores=2,
        num_subcores=16,
    )

    @functools.partial(
        pl.kernel,
        out_type=(
            jax.ShapeDtypeStruct((n_chunks, cr, e), jnp.float32),
            jax.ShapeDtypeStruct((n_chunks, cr // 4, e), jnp.float32),
            jax.ShapeDtypeStruct((n_chunks, 16, e), jnp.float32),
        ),
        mesh=mesh,
        scratch_types=(
            pltpu.VMEM((cr, e), jnp.float32),
            pltpu.VMEM((cr, e), jnp.float32),
            pltpu.VMEM((cr // 4, e), jnp.float32),
            pltpu.VMEM((16, e), jnp.float32),
        ),
    )
    def run(x_hbm, o0_hbm, o1_hbm, o2_hbm, xb, o0b, o1b, o2b):
        wid = lax.axis_index("subcore") * 2 + lax.axis_index("core")

        @pl.loop(wid, n_chunks, step=NUM_WORKERS)
        def _(t):
            pltpu.sync_copy(x_hbm.at[t], xb)
            _compute_chunk(xb, o0b, o1b, o2b)
            pltpu.sync_copy(o0b, o0_hbm.at[t])
            pltpu.sync_copy(o1b, o1_hbm.at[t])
            pltpu.sync_copy(o2b, o2_hbm.at[t])

    return run(xc)


def _decomp_sc_pipelined(xc):
    n_chunks, cr, e = xc.shape
    mesh = plsc.VectorSubcoreMesh(
        core_axis_name="core",
        subcore_axis_name="subcore",
        num_cores=2,
        num_subcores=16,
    )

    def body(xb, o0, o1, o2):
        _compute_chunk(xb.at[0], o0.at[0], o1.at[0], o2.at[0])

    @functools.partial(
        pl.kernel,
        out_type=(
            jax.ShapeDtypeStruct((n_chunks, cr, e), jnp.float32),
            jax.ShapeDtypeStruct((n_chunks, cr // 4, e), jnp.float32),
            jax.ShapeDtypeStruct((n_chunks, cr // 16, e), jnp.float32),
        ),
        mesh=mesh,
        scratch_types=(),
    )
    def run(x_hbm, o0_hbm, o1_hbm, o2_hbm):
        pltpu.emit_pipeline(
            body,
            grid=(n_chunks,),
            in_specs=[pl.BlockSpec((1, cr, e), lambda i: (i, 0, 0))],
            out_specs=[
                pl.BlockSpec((1, cr, e), lambda i: (i, 0, 0)),
                pl.BlockSpec((1, cr // 4, e), lambda i: (i, 0, 0)),
                pl.BlockSpec((1, cr // 16, e), lambda i: (i, 0, 0)),
            ],
            core_axis_name=("core", "subcore"),
            dimension_semantics=(pltpu.PARALLEL,),
        )(x_hbm, o0_hbm, o1_hbm, o2_hbm)

    return run(xc)


def _compute_chunk_l1(xb, o1, o2):
    e = xb.shape[-1]

    @pl.loop(0, GROUPS_PER_CHUNK)
    def _(g):
        r0 = g * ROWS_PER_GROUP
        for j in range(e // LANES):
            sl = pl.ds(j * LANES, LANES)
            means0 = []
            for s in range(4):
                r = [xb[r0 + 4 * s + i, sl] for i in range(4)]
                means0.append(((r[0] + r[1]) + (r[2] + r[3])) * 0.25)
            m1 = ((means0[0] + means0[1]) + (means0[2] + means0[3])) * 0.25
            for s in range(4):
                o1[g * 4 + s, sl] = means0[s] - m1
            o2[g, sl] = m1


def _level1_sc(xc):
    n_chunks, cr, e = xc.shape
    mesh = plsc.VectorSubcoreMesh(
        core_axis_name="core",
        subcore_axis_name="subcore",
        num_cores=2,
        num_subcores=16,
    )

    def body(xb, o1, o2):
        _compute_chunk_l1(xb.at[0], o1.at[0], o2.at[0])

    @functools.partial(
        pl.kernel,
        out_type=(
            jax.ShapeDtypeStruct((n_chunks, cr // 4, e), jnp.float32),
            jax.ShapeDtypeStruct((n_chunks, 16, e), jnp.float32),
        ),
        mesh=mesh,
        scratch_types=(),
    )
    def run(x_hbm, o1_hbm, o2_hbm):
        pltpu.emit_pipeline(
            body,
            grid=(n_chunks,),
            in_specs=[pl.BlockSpec((1, cr, e), lambda i: (i, 0, 0))],
            out_specs=[
                pl.BlockSpec((1, cr // 4, e), lambda i: (i, 0, 0)),
                pl.BlockSpec((1, 16, e), lambda i: (i, 0, 0)),
            ],
            core_axis_name=("core", "subcore"),
            dimension_semantics=(pltpu.PARALLEL,),
        )(x_hbm, o1_hbm, o2_hbm)

    return run(xc)


TC_BLOCK_GROUPS = 625


def _residual0_tc(xg):
    n_groups, four, e = xg.shape
    bg = TC_BLOCK_GROUPS

    def body(x_ref, o_ref):
        xs = x_ref[...]
        m0 = jnp.mean(xs, axis=1, keepdims=True)
        o_ref[...] = xs - m0

    return pl.pallas_call(
        body,
        grid=(n_groups // bg,),
        in_specs=[pl.BlockSpec((bg, four, e), lambda i: (i, 0, 0))],
        out_specs=pl.BlockSpec((bg, four, e), lambda i: (i, 0, 0)),
        out_shape=jax.ShapeDtypeStruct((n_groups, four, e), jnp.float32),
    )(xg)


@jax.jit
def kernel(x, indices_level0, indices_level1, sample_dict=0):
    b, n, e = x.shape
    n_chunks = (b * n) // CHUNK_ROWS
    xc = x.reshape(n_chunks, CHUNK_ROWS, e)
    o0, o1, o2 = _decomp_sc_pipelined(xc)
    return (
        o0.reshape(b, n, e),
        o1.reshape(b, n // 4, e),
        o2.reshape(b, n // 16, e),
    )
